# Initial kernel scaffold; baseline (speedup 1.0000x reference)
import jax
import jax.numpy as jnp
from jax.experimental import pallas as pl


def kernel(user_id, age, gender, user_location, user_time_of_day, user_day_of_week, recency, dish_id, store_id, tags, tastes, category, price, order_times, rating, item_location, item_time_of_day, item_day_of_week, user_emb_table, user_age_W, user_age_b, user_gender_table, user_location_W, user_location_b, user_time_W, user_time_b, user_day_table, user_recency_W, user_recency_b, dish_emb_table, store_emb_table, tag_emb_table, taste_emb_table, cat_emb_table, dish_price_W, dish_price_b, dish_order_times_W, dish_order_times_b, dish_rating_W, dish_rating_b, dish_location_W, dish_location_b, dish_time_W, dish_time_b, dish_day_table, user_proj_W, user_proj_b, item_proj_W, item_proj_b):
    raise NotImplementedError("write your pallas kernel here")


# speedup vs baseline: 1.8492x; 1.8492x over previous
import functools

import jax
import jax.numpy as jnp
from jax import lax
from jax.experimental import pallas as pl
from jax.experimental.pallas import tpu as pltpu
from jax.experimental.pallas import tpu_sc as plsc

_B = 16384
_D = 64
_NC = 2
_NS = 16
_NW = _NC * _NS
_PW = _B // _NW
_H = 2
_HB = _PW // _H
_C = 128

_BC = 2048


def _sc_gather_body(utab, uid, dtab, did, stab, sid, ttab, tagid, tstab,
                    tasteid, ctab, catid,
                    urows, drows, srows, tagrows, tasterows, catrows,
                    iu, idh, ist, itg, its, ict, ru, rd, rs, rt, rts, rc,
                    s1, s2, s3, s4, s5, s6):
  wid = lax.axis_index("s") * _NC + lax.axis_index("c")
  for h in range(_H):
    b = wid * _PW + h * _HB
    ics = []
    for c in range(_HB // _C):
      ics.append(pltpu.async_copy(uid.at[pl.ds(b + c * _C, _C)], iu.at[c], s1))
      ics.append(pltpu.async_copy(did.at[pl.ds(b + c * _C, _C)], idh.at[c], s2))
      ics.append(pltpu.async_copy(sid.at[pl.ds(b + c * _C, _C)], ist.at[c], s3))
      ics.append(pltpu.async_copy(catid.at[pl.ds(b + c * _C, _C)], ict.at[c], s4))
    for c in range(_HB * 5 // _C):
      ics.append(pltpu.async_copy(tagid.at[pl.ds(b * 5 + c * _C, _C)], itg.at[c], s5))
    for c in range(_HB * 3 // _C):
      ics.append(pltpu.async_copy(tasteid.at[pl.ds(b * 3 + c * _C, _C)], its.at[c], s6))
    for cp in ics:
      cp.wait()
    cps = []
    for c in range(_HB // _C):
      cps.append(pltpu.async_copy(utab.at[iu.at[c]], ru.at[pl.ds(c * _C, _C)], s1))
      cps.append(pltpu.async_copy(dtab.at[idh.at[c]], rd.at[pl.ds(c * _C, _C)], s2))
      cps.append(pltpu.async_copy(stab.at[ist.at[c]], rs.at[pl.ds(c * _C, _C)], s3))
      cps.append(pltpu.async_copy(ctab.at[ict.at[c]], rc.at[pl.ds(c * _C, _C)], s4))
    for c in range(_HB * 5 // _C):
      cps.append(pltpu.async_copy(ttab.at[itg.at[c]], rt.at[pl.ds(c * _C, _C)], s5))
    for c in range(_HB * 3 // _C):
      cps.append(pltpu.async_copy(tstab.at[its.at[c]], rts.at[pl.ds(c * _C, _C)], s6))
    for cp in cps:
      cp.wait()
    pltpu.sync_copy(ru, urows.at[pl.ds(b, _HB)])
    pltpu.sync_copy(rd, drows.at[pl.ds(b, _HB)])
    pltpu.sync_copy(rs, srows.at[pl.ds(b, _HB)])
    pltpu.sync_copy(rt, tagrows.at[pl.ds(b * 5, _HB * 5)])
    pltpu.sync_copy(rts, tasterows.at[pl.ds(b * 3, _HB * 3)])
    pltpu.sync_copy(rc, catrows.at[pl.ds(b, _HB)])


@functools.cache
def _sc_gather_kernel():
  return pl.kernel(
    _sc_gather_body,
    out_type=[
        jax.ShapeDtypeStruct((_B, 64), jnp.float32),
        jax.ShapeDtypeStruct((_B, 64), jnp.float32),
        jax.ShapeDtypeStruct((_B, 32), jnp.float32),
        jax.ShapeDtypeStruct((_B * 5, 16), jnp.float32),
        jax.ShapeDtypeStruct((_B * 3, 16), jnp.float32),
        jax.ShapeDtypeStruct((_B, 16), jnp.float32),
    ],
    mesh=plsc.VectorSubcoreMesh(core_axis_name="c", subcore_axis_name="s",
                                num_cores=_NC, num_subcores=_NS),
    scratch_types=[
        pltpu.VMEM((_HB // _C, _C), jnp.int32),
        pltpu.VMEM((_HB // _C, _C), jnp.int32),
        pltpu.VMEM((_HB // _C, _C), jnp.int32),
        pltpu.VMEM((_HB * 5 // _C, _C), jnp.int32),
        pltpu.VMEM((_HB * 3 // _C, _C), jnp.int32),
        pltpu.VMEM((_HB // _C, _C), jnp.int32),
        pltpu.VMEM((_HB, 64), jnp.float32),
        pltpu.VMEM((_HB, 64), jnp.float32),
        pltpu.VMEM((_HB, 32), jnp.float32),
        pltpu.VMEM((_HB * 5, 16), jnp.float32),
        pltpu.VMEM((_HB * 3, 16), jnp.float32),
        pltpu.VMEM((_HB, 16), jnp.float32),
        pltpu.SemaphoreType.DMA,
        pltpu.SemaphoreType.DMA,
        pltpu.SemaphoreType.DMA,
        pltpu.SemaphoreType.DMA,
        pltpu.SemaphoreType.DMA,
        pltpu.SemaphoreType.DMA,
    ],
    compiler_params=pltpu.CompilerParams(use_tc_tiling_on_sc=False),
  )


def _tc_body(urows, drows, srows, tagrows, tasterows, catrows,
             age, gender, uloc, utime, uday, rec,
             tags, tastes, price, order, rating, iloc, itime, iday,
             age_W, age_b, gender_tab, uloc_W, uloc_b, utime_W, utime_b,
             uday_tab, rec_W, rec_b,
             price_W, price_b, order_W, order_b, rating_W, rating_b,
             iloc_W, iloc_b, itime_W, itime_b, iday_tab,
             up_W, up_b, ip_W, ip_b,
             un_out, it_out, sc_out):
  f32 = jnp.float32
  Wu = up_W[...]
  Wi = ip_W[...]

  uv = jnp.dot(urows[...], Wu[0:64], preferred_element_type=f32)
  uv += age[...] * jnp.dot(age_W[...], Wu[64:80], preferred_element_type=f32)
  g1h = (gender[...] == lax.broadcasted_iota(jnp.int32, (_BC, 3), 1)).astype(f32)
  uv += jnp.dot(g1h, jnp.dot(gender_tab[...], Wu[80:96],
                             preferred_element_type=f32),
                preferred_element_type=f32)
  uv += jnp.dot(uloc[...], jnp.dot(uloc_W[...], Wu[96:112],
                                   preferred_element_type=f32),
                preferred_element_type=f32)
  uv += utime[...] * jnp.dot(utime_W[...], Wu[112:120], preferred_element_type=f32)
  ud1h = (uday[...] == lax.broadcasted_iota(jnp.int32, (_BC, 7), 1)).astype(f32)
  uv += jnp.dot(ud1h, jnp.dot(uday_tab[...], Wu[120:128],
                              preferred_element_type=f32),
                preferred_element_type=f32)
  uv += rec[...] * jnp.dot(rec_W[...], Wu[128:144], preferred_element_type=f32)
  ubias = (jnp.dot(age_b[...], Wu[64:80], preferred_element_type=f32)
           + jnp.dot(uloc_b[...], Wu[96:112], preferred_element_type=f32)
           + jnp.dot(utime_b[...], Wu[112:120], preferred_element_type=f32)
           + jnp.dot(rec_b[...], Wu[128:144], preferred_element_type=f32)
           + up_b[...])
  uv += ubias

  iv = jnp.dot(drows[...], Wi[0:64], preferred_element_type=f32)
  iv += jnp.dot(srows[...], Wi[64:96], preferred_element_type=f32)
  m_tag = (tags[...] != 0).astype(f32)
  r5 = ((lax.broadcasted_iota(jnp.int32, (5, 80), 1) // 16)
        == lax.broadcasted_iota(jnp.int32, (5, 80), 0)).astype(f32)
  s80 = ((lax.broadcasted_iota(jnp.int32, (80, 16), 0) % 16)
         == lax.broadcasted_iota(jnp.int32, (80, 16), 1)).astype(f32)
  mexp_t = jnp.dot(m_tag, r5, preferred_element_type=f32)
  inv_t = 1.0 / (jnp.sum(m_tag, axis=1, keepdims=True) + 1e-08)
  sw_t = jnp.dot(s80, Wi[96:112], preferred_element_type=f32)
  iv += jnp.dot(tagrows[...] * mexp_t * inv_t, sw_t, preferred_element_type=f32)
  m_ts = (tastes[...] != 0).astype(f32)
  r3 = ((lax.broadcasted_iota(jnp.int32, (3, 48), 1) // 16)
        == lax.broadcasted_iota(jnp.int32, (3, 48), 0)).astype(f32)
  s48 = ((lax.broadcasted_iota(jnp.int32, (48, 16), 0) % 16)
         == lax.broadcasted_iota(jnp.int32, (48, 16), 1)).astype(f32)
  mexp_s = jnp.dot(m_ts, r3, preferred_element_type=f32)
  inv_s = 1.0 / (jnp.sum(m_ts, axis=1, keepdims=True) + 1e-08)
  sw_s = jnp.dot(s48, Wi[112:128], preferred_element_type=f32)
  iv += jnp.dot(tasterows[...] * mexp_s * inv_s, sw_s, preferred_element_type=f32)
  iv += jnp.dot(catrows[...], Wi[128:144], preferred_element_type=f32)
  iv += price[...] * jnp.dot(price_W[...], Wi[144:160], preferred_element_type=f32)
  iv += order[...] * jnp.dot(order_W[...], Wi[160:168], preferred_element_type=f32)
  iv += rating[...] * jnp.dot(rating_W[...], Wi[168:176], preferred_element_type=f32)
  iv += jnp.dot(iloc[...], jnp.dot(iloc_W[...], Wi[176:192],
                                   preferred_element_type=f32),
                preferred_element_type=f32)
  iv += itime[...] * jnp.dot(itime_W[...], Wi[192:200], preferred_element_type=f32)
  id1h = (iday[...] == lax.broadcasted_iota(jnp.int32, (_BC, 7), 1)).astype(f32)
  iv += jnp.dot(id1h, jnp.dot(iday_tab[...], Wi[200:208],
                              preferred_element_type=f32),
                preferred_element_type=f32)
  ibias = (jnp.dot(price_b[...], Wi[144:160], preferred_element_type=f32)
           + jnp.dot(order_b[...], Wi[160:168], preferred_element_type=f32)
           + jnp.dot(rating_b[...], Wi[168:176], preferred_element_type=f32)
           + jnp.dot(iloc_b[...], Wi[176:192], preferred_element_type=f32)
           + jnp.dot(itime_b[...], Wi[192:200], preferred_element_type=f32)
           + ip_b[...])
  iv += ibias

  un = uv / jnp.maximum(jnp.sqrt(jnp.sum(uv * uv, axis=-1, keepdims=True)), 1e-12)
  it = iv / jnp.maximum(jnp.sqrt(jnp.sum(iv * iv, axis=-1, keepdims=True)), 1e-12)
  un_out[...] = un
  it_out[...] = it
  sc_out[...] = jnp.sum(un * it, axis=-1, keepdims=True)


def _chunk(d):
  return pl.BlockSpec((_BC, d), lambda i: (i, 0))


def _full(shape):
  return pl.BlockSpec(shape, lambda i: (0,) * len(shape))


def kernel(user_id, age, gender, user_location, user_time_of_day,
           user_day_of_week, recency, dish_id, store_id, tags, tastes,
           category, price, order_times, rating, item_location,
           item_time_of_day, item_day_of_week, user_emb_table, user_age_W,
           user_age_b, user_gender_table, user_location_W, user_location_b,
           user_time_W, user_time_b, user_day_table, user_recency_W,
           user_recency_b, dish_emb_table, store_emb_table, tag_emb_table,
           taste_emb_table, cat_emb_table, dish_price_W, dish_price_b,
           dish_order_times_W, dish_order_times_b, dish_rating_W,
           dish_rating_b, dish_location_W, dish_location_b, dish_time_W,
           dish_time_b, dish_day_table, user_proj_W, user_proj_b,
           item_proj_W, item_proj_b):
  i32 = jnp.int32
  uid2 = user_id.astype(i32)
  did2 = dish_id.astype(i32)
  sid2 = store_id.astype(i32)
  tag2 = tags.astype(i32).reshape(_B * 5)
  tas2 = tastes.astype(i32).reshape(_B * 3)
  cat2 = category.astype(i32)

  urows, drows, srows, tagrows, tasterows, catrows = _sc_gather_kernel()(
      user_emb_table, uid2, dish_emb_table, did2, store_emb_table, sid2,
      tag_emb_table, tag2, taste_emb_table, tas2, cat_emb_table, cat2)

  tagrows = tagrows.reshape(_B, 80)
  tasterows = tasterows.reshape(_B, 48)

  grid = (_B // _BC,)
  un, it, sc = pl.pallas_call(
      _tc_body,
      grid=grid,
      in_specs=[
          _chunk(64), _chunk(64), _chunk(32), _chunk(80), _chunk(48),
          _chunk(16),
          _chunk(1), _chunk(1), _chunk(2), _chunk(1), _chunk(1), _chunk(1),
          _chunk(5), _chunk(3), _chunk(1), _chunk(1), _chunk(1), _chunk(2),
          _chunk(1), _chunk(1),
          _full((1, 16)), _full((1, 16)), _full((3, 16)), _full((2, 16)),
          _full((1, 16)), _full((1, 8)), _full((1, 8)), _full((7, 8)),
          _full((1, 16)), _full((1, 16)),
          _full((1, 16)), _full((1, 16)), _full((1, 8)), _full((1, 8)),
          _full((1, 8)), _full((1, 8)), _full((2, 16)), _full((1, 16)),
          _full((1, 8)), _full((1, 8)), _full((7, 8)),
          _full((144, 64)), _full((1, 64)), _full((208, 64)), _full((1, 64)),
      ],
      out_specs=[_chunk(64), _chunk(64), _chunk(1)],
      out_shape=[
          jax.ShapeDtypeStruct((_B, 64), jnp.float32),
          jax.ShapeDtypeStruct((_B, 64), jnp.float32),
          jax.ShapeDtypeStruct((_B, 1), jnp.float32),
      ],
  )(
      urows, drows, srows, tagrows, tasterows, catrows,
      age.reshape(_B, 1), gender.astype(i32).reshape(_B, 1), user_location,
      user_time_of_day.reshape(_B, 1),
      user_day_of_week.astype(i32).reshape(_B, 1), recency.reshape(_B, 1),
      tags.astype(i32), tastes.astype(i32), price.reshape(_B, 1),
      order_times.reshape(_B, 1), rating.reshape(_B, 1), item_location,
      item_time_of_day.reshape(_B, 1),
      item_day_of_week.astype(i32).reshape(_B, 1),
      user_age_W, user_age_b.reshape(1, 16), user_gender_table,
      user_location_W, user_location_b.reshape(1, 16), user_time_W,
      user_time_b.reshape(1, 8), user_day_table, user_recency_W,
      user_recency_b.reshape(1, 16),
      dish_price_W, dish_price_b.reshape(1, 16), dish_order_times_W,
      dish_order_times_b.reshape(1, 8), dish_rating_W,
      dish_rating_b.reshape(1, 8), dish_location_W,
      dish_location_b.reshape(1, 16), dish_time_W, dish_time_b.reshape(1, 8),
      dish_day_table,
      user_proj_W, user_proj_b.reshape(1, 64), item_proj_W,
      item_proj_b.reshape(1, 64),
  )
  return (un, it, sc.reshape(_B))
